# baseline (speedup 1.0000x reference)
import functools

import jax
import jax.numpy as jnp
from jax import lax
from jax.experimental import pallas as pl
from jax.experimental.pallas import tpu as pltpu
from jax.experimental.pallas import tpu_sc as plsc

_Q = 1024
_D = 64
_K = 100000
_C = 1000
_T = 10

_QB = 128
_KB = 1024
_NKB = 98
_KPAD = _KB * _NKB

_IMAX = jnp.iinfo(jnp.int32).max
_NINF = float("-inf")


def _topk_body(y_ref, f_ref, oidx_ref, vals_ref, idx_ref):
    k = pl.program_id(1)
    nk = pl.num_programs(1)

    yb = y_ref[...]
    yn = yb / jnp.sqrt(jnp.sum(yb * yb, axis=1, keepdims=True))
    sim = lax.dot_general(
        f_ref[...], yn, (((1,), (1,)), ((), ())),
        preferred_element_type=jnp.float32)
    col = lax.broadcasted_iota(jnp.int32, (_KB, _QB), 0) + k * _KB
    work = jnp.where(col < _K, sim, _NINF)

    @pl.when(k == 0)
    def _init():
        vals_ref[...] = jnp.full((32, _QB), _NINF, jnp.float32)
        idx_ref[...] = jnp.full((32, _QB), _IMAX, jnp.int32)

    theta = vals_ref[9:10, :]
    cnt = jnp.sum((work > theta).astype(jnp.int32), axis=0, keepdims=True)
    needed = jnp.minimum(jnp.max(cnt), _T)

    @pl.when(needed > 0)
    def _extract_and_merge():
        vals_ref[10:20, :] = jnp.full((_T, _QB), _NINF, jnp.float32)
        idx_ref[10:20, :] = jnp.full((_T, _QB), _IMAX, jnp.int32)

        for t in range(_T):
            @pl.when(t < needed)
            def _round(t=t):
                if t == 0:
                    m = jnp.max(work, axis=0, keepdims=True)
                    c = jnp.min(jnp.where(work == m, col, _IMAX),
                                axis=0, keepdims=True)
                else:
                    mp = vals_ref[9 + t:10 + t, :]
                    cp = idx_ref[9 + t:10 + t, :]
                    elig = (work < mp) | ((work == mp) & (col > cp))
                    m = jnp.max(jnp.where(elig, work, _NINF),
                                axis=0, keepdims=True)
                    c = jnp.min(jnp.where(elig & (work == m), col, _IMAX),
                                axis=0, keepdims=True)
                vals_ref[10 + t:11 + t, :] = m
                idx_ref[10 + t:11 + t, :] = c

        wv = vals_ref[0:20, :]
        wi = idx_ref[0:20, :]
        nv, ni = [], []
        for _ in range(_T):
            m = jnp.max(wv, axis=0, keepdims=True)
            c = jnp.min(jnp.where(wv == m, wi, _IMAX), axis=0, keepdims=True)
            nv.append(m)
            ni.append(c)
            wv = jnp.where(wi == c, _NINF, wv)
        vals_ref[0:10, :] = jnp.concatenate(nv, axis=0)
        idx_ref[0:10, :] = jnp.concatenate(ni, axis=0)

    @pl.when(k == nk - 1)
    def _emit():
        oidx_ref[...] = jnp.concatenate(
            [idx_ref[0:10, :], jnp.zeros((6, _QB), jnp.int32)], axis=0)


def _topk_call(y_pred, fpad):
    return pl.pallas_call(
        _topk_body,
        grid=(_Q // _QB, _NKB),
        in_specs=[
            pl.BlockSpec((_QB, _D), lambda q, k: (q, 0)),
            pl.BlockSpec((_KB, _D), lambda q, k: (k, 0)),
        ],
        out_specs=pl.BlockSpec((16, _QB), lambda q, k: (q, 0)),
        out_shape=jax.ShapeDtypeStruct(((_Q // _QB) * 16, _QB), jnp.int32),
        scratch_shapes=[
            pltpu.VMEM((32, _QB), jnp.float32),
            pltpu.VMEM((32, _QB), jnp.int32),
        ],
        compiler_params=pltpu.CompilerParams(
            dimension_semantics=("parallel", "arbitrary")),
    )(y_pred, fpad)


_NC = 2
_NS = 16
_NW = _NC * _NS
_QPW = _Q // _NW
_IPW = _QPW * _T
_WORDS = _QPW * _C
_GCH = 64


def _sc_body(idx_hbm, y_hbm, out_hbm, idxv, labv, oh, sem):
    c = lax.axis_index("c")
    s = lax.axis_index("s")
    wid = s * _NC + c
    ibase = wid * _IPW
    obase = wid * _WORDS

    pltpu.sync_copy(idx_hbm.at[pl.ds(ibase, _IPW)], idxv)

    cps = []
    for j in range(_IPW // _GCH):
        cps.append(pltpu.async_copy(
            y_hbm.at[idxv.at[pl.ds(j * _GCH, _GCH)]],
            labv.at[pl.ds(j * _GCH, _GCH)], sem))
    for cp in cps:
        cp.wait()

    zero = jnp.zeros((16,), jnp.float32)

    def _zb(i, carry):
        oh[pl.ds(i * 16, 16)] = zero
        return carry

    lax.fori_loop(0, _WORDS // 16, _zb, 0)

    one = jnp.ones((16,), jnp.float32)
    for j in range(_IPW // 16):
        lab = labv[pl.ds(j * 16, 16)]
        ii = lax.iota(jnp.int32, 16) + (j * 16)
        qoff = (ii // _T) * _C
        plsc.store_scatter(oh, [qoff + lab], one)

    pltpu.sync_copy(oh, out_hbm.at[pl.ds(obase, _WORDS)])


def _sc_call(flat_idx, y):
    mesh = plsc.VectorSubcoreMesh(core_axis_name="c", subcore_axis_name="s")
    fn = functools.partial(
        pl.kernel,
        mesh=mesh,
        out_type=jax.ShapeDtypeStruct((_Q * _C,), jnp.float32),
        scratch_types=[
            pltpu.VMEM((_IPW,), jnp.int32),
            pltpu.VMEM((_IPW,), jnp.int32),
            pltpu.VMEM((_WORDS,), jnp.float32),
            pltpu.SemaphoreType.DMA,
        ],
        compiler_params=pltpu.CompilerParams(needs_layout_passes=False),
    )(_sc_body)
    return fn(flat_idx, y)


def kernel(y_pred, image_features, y):
    tidx = _topk_call(y_pred, image_features)
    tidx = tidx.reshape(_Q // _QB, 16, _QB)
    flat_idx = tidx.transpose(0, 2, 1)[:, :, :_T].reshape(-1)
    out_flat = _sc_call(flat_idx, y)
    return out_flat.reshape(_Q, _C)


# speedup vs baseline: 1.1638x; 1.1638x over previous
import functools

import jax
import jax.numpy as jnp
from jax import lax
from jax.experimental import pallas as pl
from jax.experimental.pallas import tpu as pltpu
from jax.experimental.pallas import tpu_sc as plsc

_Q = 1024
_D = 64
_K = 100000
_C = 1000
_T = 10

_QB = 128
_KB = 2048
_NKB = 49
_KPAD = _KB * _NKB

_IMAX = jnp.iinfo(jnp.int32).max
_NINF = float("-inf")


def _topk_body(y_ref, f_ref, oidx_ref, vals_ref, idx_ref):
    k = pl.program_id(1)
    nk = pl.num_programs(1)

    yb = y_ref[...]
    yn = yb / jnp.sqrt(jnp.sum(yb * yb, axis=1, keepdims=True))
    sim = lax.dot_general(
        f_ref[...], yn, (((1,), (1,)), ((), ())),
        preferred_element_type=jnp.float32)
    col = lax.broadcasted_iota(jnp.int32, (_KB, _QB), 0) + k * _KB
    work = jnp.where(col < _K, sim, _NINF)

    @pl.when(k == 0)
    def _init():
        vals_ref[...] = jnp.full((32, _QB), _NINF, jnp.float32)
        idx_ref[...] = jnp.full((32, _QB), _IMAX, jnp.int32)

    theta = vals_ref[9:10, :]
    cnt = jnp.sum((work > theta).astype(jnp.int32), axis=0, keepdims=True)
    needed = jnp.minimum(jnp.max(cnt), _T)

    @pl.when(needed > 0)
    def _extract_and_merge():
        vals_ref[10:20, :] = jnp.full((_T, _QB), _NINF, jnp.float32)
        idx_ref[10:20, :] = jnp.full((_T, _QB), _IMAX, jnp.int32)

        for t in range(_T):
            @pl.when(t < needed)
            def _round(t=t):
                if t == 0:
                    m = jnp.max(work, axis=0, keepdims=True)
                    c = jnp.min(jnp.where(work == m, col, _IMAX),
                                axis=0, keepdims=True)
                else:
                    mp = vals_ref[9 + t:10 + t, :]
                    cp = idx_ref[9 + t:10 + t, :]
                    elig = (work < mp) | ((work == mp) & (col > cp))
                    m = jnp.max(jnp.where(elig, work, _NINF),
                                axis=0, keepdims=True)
                    c = jnp.min(jnp.where(elig & (work == m), col, _IMAX),
                                axis=0, keepdims=True)
                vals_ref[10 + t:11 + t, :] = m
                idx_ref[10 + t:11 + t, :] = c

        wv = vals_ref[0:20, :]
        wi = idx_ref[0:20, :]
        nv, ni = [], []
        for _ in range(_T):
            m = jnp.max(wv, axis=0, keepdims=True)
            c = jnp.min(jnp.where(wv == m, wi, _IMAX), axis=0, keepdims=True)
            nv.append(m)
            ni.append(c)
            wv = jnp.where(wi == c, _NINF, wv)
        vals_ref[0:10, :] = jnp.concatenate(nv, axis=0)
        idx_ref[0:10, :] = jnp.concatenate(ni, axis=0)

    @pl.when(k == nk - 1)
    def _emit():
        oidx_ref[...] = jnp.concatenate(
            [idx_ref[0:10, :], jnp.zeros((6, _QB), jnp.int32)], axis=0)


def _topk_call(y_pred, fpad):
    return pl.pallas_call(
        _topk_body,
        grid=(_Q // _QB, _NKB),
        in_specs=[
            pl.BlockSpec((_QB, _D), lambda q, k: (q, 0)),
            pl.BlockSpec((_KB, _D), lambda q, k: (k, 0)),
        ],
        out_specs=pl.BlockSpec((16, _QB), lambda q, k: (q, 0)),
        out_shape=jax.ShapeDtypeStruct(((_Q // _QB) * 16, _QB), jnp.int32),
        scratch_shapes=[
            pltpu.VMEM((32, _QB), jnp.float32),
            pltpu.VMEM((32, _QB), jnp.int32),
        ],
        compiler_params=pltpu.CompilerParams(
            dimension_semantics=("parallel", "arbitrary")),
    )(y_pred, fpad)


_NC = 2
_NS = 16
_NW = _NC * _NS
_QPW = _Q // _NW
_IPW = _QPW * _T
_WORDS = _QPW * _C
_GCH = 64


def _sc_body(idx_hbm, y_hbm, out_hbm, idxv, labv, oh, sem):
    c = lax.axis_index("c")
    s = lax.axis_index("s")
    wid = s * _NC + c
    ibase = wid * _IPW
    obase = wid * _WORDS

    pltpu.sync_copy(idx_hbm.at[pl.ds(ibase, _IPW)], idxv)

    cps = []
    for j in range(_IPW // _GCH):
        cps.append(pltpu.async_copy(
            y_hbm.at[idxv.at[pl.ds(j * _GCH, _GCH)]],
            labv.at[pl.ds(j * _GCH, _GCH)], sem))
    for cp in cps:
        cp.wait()

    zero = jnp.zeros((16,), jnp.float32)

    def _zb(i, carry):
        oh[pl.ds(i * 16, 16)] = zero
        return carry

    lax.fori_loop(0, _WORDS // 16, _zb, 0)

    one = jnp.ones((16,), jnp.float32)
    for j in range(_IPW // 16):
        lab = labv[pl.ds(j * 16, 16)]
        ii = lax.iota(jnp.int32, 16) + (j * 16)
        qoff = (ii // _T) * _C
        plsc.store_scatter(oh, [qoff + lab], one)

    pltpu.sync_copy(oh, out_hbm.at[pl.ds(obase, _WORDS)])


def _sc_call(flat_idx, y):
    mesh = plsc.VectorSubcoreMesh(core_axis_name="c", subcore_axis_name="s")
    fn = functools.partial(
        pl.kernel,
        mesh=mesh,
        out_type=jax.ShapeDtypeStruct((_Q * _C,), jnp.float32),
        scratch_types=[
            pltpu.VMEM((_IPW,), jnp.int32),
            pltpu.VMEM((_IPW,), jnp.int32),
            pltpu.VMEM((_WORDS,), jnp.float32),
            pltpu.SemaphoreType.DMA,
        ],
        compiler_params=pltpu.CompilerParams(needs_layout_passes=False),
    )(_sc_body)
    return fn(flat_idx, y)


def kernel(y_pred, image_features, y):
    tidx = _topk_call(y_pred, image_features)
    tidx = tidx.reshape(_Q // _QB, 16, _QB)
    flat_idx = tidx.transpose(0, 2, 1)[:, :, :_T].reshape(-1)
    out_flat = _sc_call(flat_idx, y)
    return out_flat.reshape(_Q, _C)
